# baseline (speedup 1.0000x reference)
import functools

import jax
import jax.numpy as jnp
from jax import lax
from jax.experimental import pallas as pl
from jax.experimental.pallas import tpu as pltpu
from jax.experimental.pallas import tpu_sc as plsc

V = 1000000
D = 64
B = 16384
NN = 5
NW = 32
BPW = B // NW
NCH = BPW // 128


def _sc_kernel():
    mesh = plsc.VectorSubcoreMesh(core_axis_name="c", subcore_axis_name="s")

    @functools.partial(
        pl.kernel,
        mesh=mesh,
        compiler_params=pltpu.CompilerParams(
            needs_layout_passes=False, use_tc_tiling_on_sc=False),
        out_type=jax.ShapeDtypeStruct((NN, B // 128, 128), jnp.float32),
        scratch_types=[
            pltpu.VMEM((NCH, 128), jnp.int32),
            pltpu.VMEM((BPW, D), jnp.float32),
            pltpu.VMEM((BPW, D), jnp.float32),
            pltpu.VMEM((NCH, 128), jnp.float32),
            pltpu.SemaphoreType.DMA,
        ],
    )
    def k(tgt_hbm, ctx_hbm, wt_hbm, wc_hbm, out_hbm, idx_v, rows_t, rows_c,
          dots_v, sem):
        wid = lax.axis_index("s") * 2 + lax.axis_index("c")
        base = wid * BPW
        crow = wid * NCH

        def gather(table, idx_ref, rows_ref):
            for j in range(NCH):
                pltpu.async_copy(
                    table.at[idx_ref.at[j]],
                    rows_ref.at[pl.ds(j * 128, 128)], sem)
            for j in range(NCH):
                pltpu.make_async_copy(
                    table.at[idx_ref.at[0]],
                    rows_ref.at[pl.ds(0, 128)], sem).wait()

        pltpu.sync_copy(tgt_hbm.at[pl.ds(crow, NCH)], idx_v)
        gather(wt_hbm, idx_v, rows_t)

        lanes = lax.iota(jnp.int32, 16)

        def dot_body(g, _):
            res = jnp.zeros((16,), jnp.float32)
            for i in range(16):
                b = g * 16 + i
                p = None
                for dc in range(D // 16):
                    we = rows_t[b, pl.ds(dc * 16, 16)]
                    ce = rows_c[b, pl.ds(dc * 16, 16)]
                    p = we * ce if p is None else p + we * ce
                res = jnp.where(lanes == i, jnp.sum(p), res)
            dots_v[g // 8, pl.ds((g % 8) * 16, 16)] = res
            return _

        for n in range(NN):
            pltpu.sync_copy(ctx_hbm.at[n, pl.ds(crow, NCH)], idx_v)
            gather(wc_hbm, idx_v, rows_c)
            lax.fori_loop(0, BPW // 16, dot_body, 0)
            pltpu.sync_copy(dots_v, out_hbm.at[n, pl.ds(crow, NCH)])

    return k


_k = _sc_kernel()


def kernel(target, context, W_target, W_context):
    tgt2 = target.reshape(B // 128, 128)
    ctx3 = context.reshape(B, NN).T.reshape(NN, B // 128, 128)
    out = _k(tgt2, ctx3, W_target, W_context)
    return out.reshape(NN, B).T
